# baseline (speedup 1.0000x reference)
import jax
import jax.numpy as jnp
from jax import lax
from jax.experimental import pallas as pl
from jax.experimental.pallas import tpu as pltpu
from jax.experimental.pallas import tpu_sc as plsc

BATCH = 16384
EMBED = 64
H1 = 128
NUM_ROWS = 1000000
NC = 2
NS = 16
NW = NC * NS
B_PER_W = BATCH // NW
IDX_CHUNK = 128
N_CHUNKS = B_PER_W // IDX_CHUNK

BLKN = 8192
G_GRID = (NUM_ROWS + BLKN - 1) // BLKN


def _g_matmul_kernel(t_ref, w_ref, g_ref):
    g = jax.lax.dot_general(
        t_ref[...], w_ref[...],
        dimension_numbers=(((0,), (0,)), ((), ())),
        preferred_element_type=jnp.float32,
        precision=jax.lax.Precision.DEFAULT,
    )
    g_ref[...] = pltpu.bitcast(g.astype(jnp.bfloat16), jnp.int32)


def _g_matmul(t_t, w):
    return pl.pallas_call(
        _g_matmul_kernel,
        grid=(G_GRID,),
        in_specs=[
            pl.BlockSpec((EMBED, BLKN), lambda i: (0, i)),
            pl.BlockSpec((EMBED, H1), lambda i: (0, 0)),
        ],
        out_specs=pl.BlockSpec((BLKN // 2, H1), lambda i: (i, 0)),
        out_shape=jax.ShapeDtypeStruct((NUM_ROWS // 2, H1), jnp.int32),
    )(t_t, w)


def _sc_gather_kernel(gu_hbm, gv_hbm, uidx_hbm, iidx_hbm, hu_hbm, hv_hbm,
                      uidx_v, iidx_v, urows_v, irows_v, sem):
    wid = lax.axis_index("s") * NC + lax.axis_index("c")
    pltpu.sync_copy(uidx_hbm.at[wid], uidx_v)
    pltpu.sync_copy(iidx_hbm.at[wid], iidx_v)
    base = wid * B_PER_W
    for c in range(N_CHUNKS):
        cu = pltpu.async_copy(gu_hbm.at[uidx_v.at[c]], urows_v, sem)
        ci = pltpu.async_copy(gv_hbm.at[iidx_v.at[c]], irows_v, sem)
        cu.wait()
        pltpu.sync_copy(urows_v,
                        hu_hbm.at[pl.ds(base + c * IDX_CHUNK, IDX_CHUNK)])
        ci.wait()
        pltpu.sync_copy(irows_v,
                        hv_hbm.at[pl.ds(base + c * IDX_CHUNK, IDX_CHUNK)])


def _sc_gather(gu, gv, user_ids, item_ids):
    mesh = plsc.VectorSubcoreMesh(core_axis_name="c", subcore_axis_name="s")
    uidx = (user_ids.astype(jnp.int32) >> 1).reshape(NW, N_CHUNKS, IDX_CHUNK)
    iidx = (item_ids.astype(jnp.int32) >> 1).reshape(NW, N_CHUNKS, IDX_CHUNK)
    out_sds = jax.ShapeDtypeStruct((BATCH, H1), jnp.int32)
    k = pl.kernel(
        _sc_gather_kernel,
        out_type=(out_sds, out_sds),
        mesh=mesh,
        scratch_types=[
            pltpu.VMEM((N_CHUNKS, IDX_CHUNK), jnp.int32),
            pltpu.VMEM((N_CHUNKS, IDX_CHUNK), jnp.int32),
            pltpu.VMEM((IDX_CHUNK, H1), jnp.int32),
            pltpu.VMEM((IDX_CHUNK, H1), jnp.int32),
            pltpu.SemaphoreType.DMA,
        ],
    )
    return k(gu, gv, uidx, iidx)


BLK = 2048


def _unpack(packed_ref, parity_ref):
    u = lax.bitcast_convert_type(packed_ref[...], jnp.uint32)
    lo = lax.bitcast_convert_type(u << 16, jnp.float32)
    hi = lax.bitcast_convert_type(u & jnp.uint32(0xFFFF0000), jnp.float32)
    return jnp.where(parity_ref[...] > 0.5, hi, lo)


def _mlp_kernel(hu_ref, hv_ref, up_ref, vp_ref, b1_ref, w2_ref, b2_ref,
                w3_ref, b3_ref, w4t_ref, b4_ref, o_ref):
    h = _unpack(hu_ref, up_ref) + _unpack(hv_ref, vp_ref) + b1_ref[...]
    h = jnp.maximum(h, 0.0)
    h = jnp.maximum(h @ w2_ref[...] + b2_ref[...], 0.0)
    h = jnp.maximum(h @ w3_ref[...] + b3_ref[...], 0.0)
    logit = jnp.sum(h * w4t_ref[...], axis=1, keepdims=True) + b4_ref[...]
    o_ref[...] = jax.nn.sigmoid(logit)


def _mlp(hu, hv, up, vp, b1, W2, b2, W3, b3, W4, b4):
    full = lambda shape: pl.BlockSpec(shape, lambda i: (0, 0))
    out = pl.pallas_call(
        _mlp_kernel,
        grid=(BATCH // BLK,),
        in_specs=[
            pl.BlockSpec((BLK, H1), lambda i: (i, 0)),
            pl.BlockSpec((BLK, H1), lambda i: (i, 0)),
            pl.BlockSpec((BLK, 1), lambda i: (i, 0)),
            pl.BlockSpec((BLK, 1), lambda i: (i, 0)),
            full((1, 128)),
            full(W2.shape), full((1, 64)),
            full(W3.shape), full((1, 32)),
            full((1, 32)), full((1, 1)),
        ],
        out_specs=pl.BlockSpec((BLK, 1), lambda i: (i, 0)),
        out_shape=jax.ShapeDtypeStruct((BATCH, 1), jnp.float32),
    )(hu, hv, up, vp, b1.reshape(1, -1),
      W2, b2.reshape(1, -1), W3, b3.reshape(1, -1),
      W4.reshape(1, -1), b4.reshape(1, 1))
    return out.reshape(BATCH)


def kernel(user_ids, item_ids, user_table, item_table,
           W1, b1, W2, b2, W3, b3, W4, b4):
    gu = _g_matmul(user_table.T, W1[:EMBED])
    gv = _g_matmul(item_table.T, W1[EMBED:])
    hu, hv = _sc_gather(gu, gv, user_ids, item_ids)
    up = (user_ids & 1).astype(jnp.float32).reshape(BATCH, 1)
    vp = (item_ids & 1).astype(jnp.float32).reshape(BATCH, 1)
    return _mlp(hu, hv, up, vp, b1, W2, b2, W3, b3, W4, b4)


# speedup vs baseline: 1.3186x; 1.3186x over previous
import jax
import jax.numpy as jnp
from jax import lax
from jax.experimental import pallas as pl
from jax.experimental.pallas import tpu as pltpu
from jax.experimental.pallas import tpu_sc as plsc

BATCH = 16384
EMBED = 64
H1 = 128
NUM_ROWS = 1000000
NC = 2
NS = 16
NW = NC * NS
B_PER_W = BATCH // NW
IDX_CHUNK = 128
N_CHUNKS = B_PER_W // IDX_CHUNK

BLKN = 16384
G_GRID = (NUM_ROWS + BLKN - 1) // BLKN


def _g_matmul_kernel(t_ref, w_ref, g_ref):
    g = jax.lax.dot_general(
        t_ref[...].astype(jnp.bfloat16), w_ref[...].astype(jnp.bfloat16),
        dimension_numbers=(((0,), (0,)), ((), ())),
        preferred_element_type=jnp.float32,
    )
    g_ref[...] = pltpu.bitcast(g.astype(jnp.bfloat16), jnp.int32)


def _g_matmul(t_t, w):
    return pl.pallas_call(
        _g_matmul_kernel,
        grid=(G_GRID,),
        in_specs=[
            pl.BlockSpec((EMBED, BLKN), lambda i: (0, i)),
            pl.BlockSpec((EMBED, H1), lambda i: (0, 0)),
        ],
        out_specs=pl.BlockSpec((BLKN // 2, H1), lambda i: (i, 0)),
        out_shape=jax.ShapeDtypeStruct((NUM_ROWS // 2, H1), jnp.int32),
    )(t_t, w)


def _sc_gather_kernel(gu_hbm, gv_hbm, uidx_hbm, iidx_hbm, hu_hbm, hv_hbm,
                      uidx_v, iidx_v, urows_v, irows_v, sem):
    wid = lax.axis_index("s") * NC + lax.axis_index("c")
    pltpu.sync_copy(uidx_hbm.at[wid], uidx_v)
    pltpu.sync_copy(iidx_hbm.at[wid], iidx_v)
    base = wid * B_PER_W
    for c in range(N_CHUNKS):
        cu = pltpu.async_copy(gu_hbm.at[uidx_v.at[c]], urows_v, sem)
        ci = pltpu.async_copy(gv_hbm.at[iidx_v.at[c]], irows_v, sem)
        cu.wait()
        pltpu.sync_copy(urows_v,
                        hu_hbm.at[pl.ds(base + c * IDX_CHUNK, IDX_CHUNK)])
        ci.wait()
        pltpu.sync_copy(irows_v,
                        hv_hbm.at[pl.ds(base + c * IDX_CHUNK, IDX_CHUNK)])


def _sc_gather(gu, gv, user_ids, item_ids):
    mesh = plsc.VectorSubcoreMesh(core_axis_name="c", subcore_axis_name="s")
    uidx = (user_ids.astype(jnp.int32) >> 1).reshape(NW, N_CHUNKS, IDX_CHUNK)
    iidx = (item_ids.astype(jnp.int32) >> 1).reshape(NW, N_CHUNKS, IDX_CHUNK)
    out_sds = jax.ShapeDtypeStruct((BATCH, H1), jnp.int32)
    k = pl.kernel(
        _sc_gather_kernel,
        out_type=(out_sds, out_sds),
        mesh=mesh,
        scratch_types=[
            pltpu.VMEM((N_CHUNKS, IDX_CHUNK), jnp.int32),
            pltpu.VMEM((N_CHUNKS, IDX_CHUNK), jnp.int32),
            pltpu.VMEM((IDX_CHUNK, H1), jnp.int32),
            pltpu.VMEM((IDX_CHUNK, H1), jnp.int32),
            pltpu.SemaphoreType.DMA,
        ],
    )
    return k(gu, gv, uidx, iidx)


BLK = 2048


def _unpack(packed_ref, parity_ref):
    u = lax.bitcast_convert_type(packed_ref[...], jnp.uint32)
    lo = lax.bitcast_convert_type(u << 16, jnp.float32)
    hi = lax.bitcast_convert_type(u & jnp.uint32(0xFFFF0000), jnp.float32)
    return jnp.where(parity_ref[...] > 0.5, hi, lo)


def _mlp_kernel(hu_ref, hv_ref, up_ref, vp_ref, b1_ref, w2_ref, b2_ref,
                w3_ref, b3_ref, w4t_ref, b4_ref, o_ref):
    h = _unpack(hu_ref, up_ref) + _unpack(hv_ref, vp_ref) + b1_ref[...]
    h = jnp.maximum(h, 0.0)
    h = jnp.maximum(h @ w2_ref[...] + b2_ref[...], 0.0)
    h = jnp.maximum(h @ w3_ref[...] + b3_ref[...], 0.0)
    logit = jnp.sum(h * w4t_ref[...], axis=1, keepdims=True) + b4_ref[...]
    o_ref[...] = jax.nn.sigmoid(logit)


def _mlp(hu, hv, up, vp, b1, W2, b2, W3, b3, W4, b4):
    full = lambda shape: pl.BlockSpec(shape, lambda i: (0, 0))
    out = pl.pallas_call(
        _mlp_kernel,
        grid=(BATCH // BLK,),
        in_specs=[
            pl.BlockSpec((BLK, H1), lambda i: (i, 0)),
            pl.BlockSpec((BLK, H1), lambda i: (i, 0)),
            pl.BlockSpec((BLK, 1), lambda i: (i, 0)),
            pl.BlockSpec((BLK, 1), lambda i: (i, 0)),
            full((1, 128)),
            full(W2.shape), full((1, 64)),
            full(W3.shape), full((1, 32)),
            full((1, 32)), full((1, 1)),
        ],
        out_specs=pl.BlockSpec((BLK, 1), lambda i: (i, 0)),
        out_shape=jax.ShapeDtypeStruct((BATCH, 1), jnp.float32),
    )(hu, hv, up, vp, b1.reshape(1, -1),
      W2, b2.reshape(1, -1), W3, b3.reshape(1, -1),
      W4.reshape(1, -1), b4.reshape(1, 1))
    return out.reshape(BATCH)


def kernel(user_ids, item_ids, user_table, item_table,
           W1, b1, W2, b2, W3, b3, W4, b4):
    gu = _g_matmul(user_table.T, W1[:EMBED])
    gv = _g_matmul(item_table.T, W1[EMBED:])
    hu, hv = _sc_gather(gu, gv, user_ids, item_ids)
    up = (user_ids & 1).astype(jnp.float32).reshape(BATCH, 1)
    vp = (item_ids & 1).astype(jnp.float32).reshape(BATCH, 1)
    return _mlp(hu, hv, up, vp, b1, W2, b2, W3, b3, W4, b4)


# speedup vs baseline: 1.3662x; 1.0361x over previous
import jax
import jax.numpy as jnp
from jax import lax
from jax.experimental import pallas as pl
from jax.experimental.pallas import tpu as pltpu
from jax.experimental.pallas import tpu_sc as plsc

BATCH = 16384
EMBED = 64
H1 = 128
NUM_ROWS = 1000000
NC = 2
NS = 16
NW = NC * NS
B_PER_W = BATCH // NW
IDX_CHUNK = 128
N_CHUNKS = B_PER_W // IDX_CHUNK

BLKN = 32768
G_GRID = (NUM_ROWS + BLKN - 1) // BLKN


def _g_matmul_kernel(t_ref, w_ref, g_ref):
    g = jax.lax.dot_general(
        t_ref[...].astype(jnp.bfloat16), w_ref[...].astype(jnp.bfloat16),
        dimension_numbers=(((0,), (0,)), ((), ())),
        preferred_element_type=jnp.float32,
    )
    g_ref[...] = pltpu.bitcast(g.astype(jnp.bfloat16), jnp.int32)


def _g_matmul(t_t, w):
    return pl.pallas_call(
        _g_matmul_kernel,
        grid=(G_GRID,),
        in_specs=[
            pl.BlockSpec((EMBED, BLKN), lambda i: (0, i)),
            pl.BlockSpec((EMBED, H1), lambda i: (0, 0)),
        ],
        out_specs=pl.BlockSpec((BLKN // 2, H1), lambda i: (i, 0)),
        out_shape=jax.ShapeDtypeStruct((NUM_ROWS // 2, H1), jnp.int32),
    )(t_t, w)


def _sc_gather_kernel(gu_hbm, gv_hbm, uidx_hbm, iidx_hbm, hu_hbm, hv_hbm,
                      uidx_v, iidx_v, urows_v, irows_v, sem):
    wid = lax.axis_index("s") * NC + lax.axis_index("c")
    pltpu.sync_copy(uidx_hbm.at[wid], uidx_v)
    pltpu.sync_copy(iidx_hbm.at[wid], iidx_v)
    base = wid * B_PER_W
    for c in range(N_CHUNKS):
        cu = pltpu.async_copy(gu_hbm.at[uidx_v.at[c]], urows_v, sem)
        ci = pltpu.async_copy(gv_hbm.at[iidx_v.at[c]], irows_v, sem)
        cu.wait()
        pltpu.sync_copy(urows_v,
                        hu_hbm.at[pl.ds(base + c * IDX_CHUNK, IDX_CHUNK)])
        ci.wait()
        pltpu.sync_copy(irows_v,
                        hv_hbm.at[pl.ds(base + c * IDX_CHUNK, IDX_CHUNK)])


def _sc_gather(gu, gv, user_ids, item_ids):
    mesh = plsc.VectorSubcoreMesh(core_axis_name="c", subcore_axis_name="s")
    uidx = (user_ids.astype(jnp.int32) >> 1).reshape(NW, N_CHUNKS, IDX_CHUNK)
    iidx = (item_ids.astype(jnp.int32) >> 1).reshape(NW, N_CHUNKS, IDX_CHUNK)
    out_sds = jax.ShapeDtypeStruct((BATCH, H1), jnp.int32)
    k = pl.kernel(
        _sc_gather_kernel,
        out_type=(out_sds, out_sds),
        mesh=mesh,
        scratch_types=[
            pltpu.VMEM((N_CHUNKS, IDX_CHUNK), jnp.int32),
            pltpu.VMEM((N_CHUNKS, IDX_CHUNK), jnp.int32),
            pltpu.VMEM((IDX_CHUNK, H1), jnp.int32),
            pltpu.VMEM((IDX_CHUNK, H1), jnp.int32),
            pltpu.SemaphoreType.DMA,
        ],
    )
    return k(gu, gv, uidx, iidx)


BLK = 2048


def _unpack(packed_ref, parity_ref):
    u = lax.bitcast_convert_type(packed_ref[...], jnp.uint32)
    lo = lax.bitcast_convert_type(u << 16, jnp.float32)
    hi = lax.bitcast_convert_type(u & jnp.uint32(0xFFFF0000), jnp.float32)
    return jnp.where(parity_ref[...] > 0.5, hi, lo)


def _mlp_kernel(hu_ref, hv_ref, up_ref, vp_ref, b1_ref, w2_ref, b2_ref,
                w3_ref, b3_ref, w4t_ref, b4_ref, o_ref):
    h = _unpack(hu_ref, up_ref) + _unpack(hv_ref, vp_ref) + b1_ref[...]
    h = jnp.maximum(h, 0.0)
    h = jnp.maximum(h @ w2_ref[...] + b2_ref[...], 0.0)
    h = jnp.maximum(h @ w3_ref[...] + b3_ref[...], 0.0)
    logit = jnp.sum(h * w4t_ref[...], axis=1, keepdims=True) + b4_ref[...]
    o_ref[...] = jax.nn.sigmoid(logit)


def _mlp(hu, hv, up, vp, b1, W2, b2, W3, b3, W4, b4):
    full = lambda shape: pl.BlockSpec(shape, lambda i: (0, 0))
    out = pl.pallas_call(
        _mlp_kernel,
        grid=(BATCH // BLK,),
        in_specs=[
            pl.BlockSpec((BLK, H1), lambda i: (i, 0)),
            pl.BlockSpec((BLK, H1), lambda i: (i, 0)),
            pl.BlockSpec((BLK, 1), lambda i: (i, 0)),
            pl.BlockSpec((BLK, 1), lambda i: (i, 0)),
            full((1, 128)),
            full(W2.shape), full((1, 64)),
            full(W3.shape), full((1, 32)),
            full((1, 32)), full((1, 1)),
        ],
        out_specs=pl.BlockSpec((BLK, 1), lambda i: (i, 0)),
        out_shape=jax.ShapeDtypeStruct((BATCH, 1), jnp.float32),
    )(hu, hv, up, vp, b1.reshape(1, -1),
      W2, b2.reshape(1, -1), W3, b3.reshape(1, -1),
      W4.reshape(1, -1), b4.reshape(1, 1))
    return out.reshape(BATCH)


def kernel(user_ids, item_ids, user_table, item_table,
           W1, b1, W2, b2, W3, b3, W4, b4):
    gu = _g_matmul(user_table.T, W1[:EMBED])
    gv = _g_matmul(item_table.T, W1[EMBED:])
    hu, hv = _sc_gather(gu, gv, user_ids, item_ids)
    up = (user_ids & 1).astype(jnp.float32).reshape(BATCH, 1)
    vp = (item_ids & 1).astype(jnp.float32).reshape(BATCH, 1)
    return _mlp(hu, hv, up, vp, b1, W2, b2, W3, b3, W4, b4)


# speedup vs baseline: 1.3761x; 1.0073x over previous
import jax
import jax.numpy as jnp
from jax import lax
from jax.experimental import pallas as pl
from jax.experimental.pallas import tpu as pltpu
from jax.experimental.pallas import tpu_sc as plsc

BATCH = 16384
EMBED = 64
H1 = 128
NUM_ROWS = 1000000
NC = 2
NS = 16
NW = NC * NS
B_PER_W = BATCH // NW
IDX_CHUNK = 128
N_CHUNKS = B_PER_W // IDX_CHUNK

BLKN = 32768
G_GRID = (NUM_ROWS + BLKN - 1) // BLKN


def _g_matmul_kernel(t_ref, w_ref, g_ref):
    g = jax.lax.dot_general(
        t_ref[...].astype(jnp.bfloat16), w_ref[...].astype(jnp.bfloat16),
        dimension_numbers=(((0,), (0,)), ((), ())),
        preferred_element_type=jnp.float32,
    )
    g_ref[...] = pltpu.bitcast(g.astype(jnp.bfloat16), jnp.int32)


def _g_matmul(t_t, w):
    return pl.pallas_call(
        _g_matmul_kernel,
        grid=(G_GRID,),
        in_specs=[
            pl.BlockSpec((EMBED, BLKN), lambda i: (0, i)),
            pl.BlockSpec((EMBED, H1), lambda i: (0, 0)),
        ],
        out_specs=pl.BlockSpec((BLKN // 2, H1), lambda i: (i, 0)),
        out_shape=jax.ShapeDtypeStruct((NUM_ROWS // 2, H1), jnp.int32),
    )(t_t, w)


def _sc_gather_kernel(g_hbm, idx_hbm, h_hbm, idx_v, rows_v, rows2_v, sem):
    wid = lax.axis_index("s") * NC + lax.axis_index("c")
    pltpu.sync_copy(idx_hbm.at[wid], idx_v)
    base = wid * B_PER_W
    for c in range(0, N_CHUNKS, 2):
        c0 = pltpu.async_copy(g_hbm.at[idx_v.at[c]], rows_v, sem)
        c1 = pltpu.async_copy(g_hbm.at[idx_v.at[c + 1]], rows2_v, sem)
        c0.wait()
        pltpu.sync_copy(rows_v,
                        h_hbm.at[pl.ds(base + c * IDX_CHUNK, IDX_CHUNK)])
        c1.wait()
        pltpu.sync_copy(rows2_v,
                        h_hbm.at[pl.ds(base + (c + 1) * IDX_CHUNK, IDX_CHUNK)])


def _sc_gather(g, ids):
    mesh = plsc.VectorSubcoreMesh(core_axis_name="c", subcore_axis_name="s")
    idx = (ids.astype(jnp.int32) >> 1).reshape(NW, N_CHUNKS, IDX_CHUNK)
    k = pl.kernel(
        _sc_gather_kernel,
        out_type=jax.ShapeDtypeStruct((BATCH, H1), jnp.int32),
        mesh=mesh,
        scratch_types=[
            pltpu.VMEM((N_CHUNKS, IDX_CHUNK), jnp.int32),
            pltpu.VMEM((IDX_CHUNK, H1), jnp.int32),
            pltpu.VMEM((IDX_CHUNK, H1), jnp.int32),
            pltpu.SemaphoreType.DMA,
        ],
    )
    return k(g, idx)


BLK = 2048


def _unpack(packed_ref, parity_ref):
    u = lax.bitcast_convert_type(packed_ref[...], jnp.uint32)
    lo = lax.bitcast_convert_type(u << 16, jnp.float32)
    hi = lax.bitcast_convert_type(u & jnp.uint32(0xFFFF0000), jnp.float32)
    return jnp.where(parity_ref[...] > 0.5, hi, lo)


def _mlp_kernel(hu_ref, hv_ref, up_ref, vp_ref, b1_ref, w2_ref, b2_ref,
                w3_ref, b3_ref, w4t_ref, b4_ref, o_ref):
    h = _unpack(hu_ref, up_ref) + _unpack(hv_ref, vp_ref) + b1_ref[...]
    h = jnp.maximum(h, 0.0)
    h = jnp.maximum(h @ w2_ref[...] + b2_ref[...], 0.0)
    h = jnp.maximum(h @ w3_ref[...] + b3_ref[...], 0.0)
    logit = jnp.sum(h * w4t_ref[...], axis=1, keepdims=True) + b4_ref[...]
    o_ref[...] = jax.nn.sigmoid(logit)


def _mlp(hu, hv, up, vp, b1, W2, b2, W3, b3, W4, b4):
    full = lambda shape: pl.BlockSpec(shape, lambda i: (0, 0))
    out = pl.pallas_call(
        _mlp_kernel,
        grid=(BATCH // BLK,),
        in_specs=[
            pl.BlockSpec((BLK, H1), lambda i: (i, 0)),
            pl.BlockSpec((BLK, H1), lambda i: (i, 0)),
            pl.BlockSpec((BLK, 1), lambda i: (i, 0)),
            pl.BlockSpec((BLK, 1), lambda i: (i, 0)),
            full((1, 128)),
            full(W2.shape), full((1, 64)),
            full(W3.shape), full((1, 32)),
            full((1, 32)), full((1, 1)),
        ],
        out_specs=pl.BlockSpec((BLK, 1), lambda i: (i, 0)),
        out_shape=jax.ShapeDtypeStruct((BATCH, 1), jnp.float32),
    )(hu, hv, up, vp, b1.reshape(1, -1),
      W2, b2.reshape(1, -1), W3, b3.reshape(1, -1),
      W4.reshape(1, -1), b4.reshape(1, 1))
    return out.reshape(BATCH)


def kernel(user_ids, item_ids, user_table, item_table,
           W1, b1, W2, b2, W3, b3, W4, b4):
    gu = _g_matmul(user_table.T, W1[:EMBED])
    hu = _sc_gather(gu, user_ids)
    gv = _g_matmul(item_table.T, W1[EMBED:])
    hv = _sc_gather(gv, item_ids)
    up = (user_ids & 1).astype(jnp.float32).reshape(BATCH, 1)
    vp = (item_ids & 1).astype(jnp.float32).reshape(BATCH, 1)
    return _mlp(hu, hv, up, vp, b1, W2, b2, W3, b3, W4, b4)
